# Initial kernel scaffold; baseline (speedup 1.0000x reference)
import jax
import jax.numpy as jnp
from jax.experimental import pallas as pl


def kernel(x, edge_index, W_lin1, b_lin1, W_self, W_neigh, b_layers, W_cls, b_cls):
    raise NotImplementedError("write your pallas kernel here")


# speedup vs baseline: 6.1847x; 6.1847x over previous
import jax
import jax.numpy as jnp
from jax import lax
from jax.experimental import pallas as pl
from jax.experimental.pallas import tpu as pltpu
from jax.experimental.pallas import tpu_sc as plsc

_N = 10000
_E = 320000
_H = 128
_NC = 2
_NS = 16
_NW = _NC * _NS
_EPW = _E // _NW
_B = 80
_NSTEP = _EPW // _B
_NPAD = 10240
_RPT = _NPAD // _NS
_DW = 128
_ROWBLK = 1000
_GRID = _N // _ROWBLK

_mesh = plsc.VectorSubcoreMesh(
    core_axis_name="c", subcore_axis_name="s",
    num_cores=_NC, num_subcores=_NS)


def _agg_body(hn, srcs, dsts, zrows, out, src_v, dst_v, rows_v, acc, sem):
    c = lax.axis_index("c")
    s = lax.axis_index("s")
    wid = c * _NS + s
    base = s * _RPT

    pltpu.sync_copy(zrows, acc.at[pl.ds(base, _RPT)])
    pltpu.sync_copy(srcs.at[wid], src_v)
    pltpu.sync_copy(dsts.at[wid], dst_v)
    plsc.subcore_barrier()

    def step(i, carry):
        pltpu.async_copy(hn.at[src_v.at[i]], rows_v, sem).wait()
        pltpu.sync_copy(rows_v, acc.at[dst_v.at[i]], add=True)
        return carry

    lax.fori_loop(0, _NSTEP, step, 0)
    plsc.subcore_barrier()

    pltpu.sync_copy(acc.at[pl.ds(base, _RPT)], out.at[c, pl.ds(base, _RPT)])


_sc_agg = pl.kernel(
    _agg_body,
    out_type=jax.ShapeDtypeStruct((_NC, _NPAD, _H), jnp.float32),
    mesh=_mesh,
    scratch_types=[
        pltpu.VMEM((_NSTEP, _B), jnp.int32),
        pltpu.VMEM((_NSTEP, _B), jnp.int32),
        pltpu.VMEM((_B, _H), jnp.float32),
        pltpu.VMEM_SHARED((_NPAD, _H), jnp.float32),
        pltpu.SemaphoreType.DMA,
    ],
)


def _deg_body(dsts, dz, ones_in, degout, dst_v, ones_v, dacc):
    c = lax.axis_index("c")
    s = lax.axis_index("s")
    wid = c * _NS + s
    base = s * _RPT

    pltpu.sync_copy(dz, dacc.at[pl.ds(base, _RPT)])
    pltpu.sync_copy(ones_in, ones_v)
    pltpu.sync_copy(dsts.at[wid], dst_v)
    plsc.subcore_barrier()

    def step(i, carry):
        pltpu.sync_copy(ones_v, dacc.at[dst_v.at[i]], add=True)
        return carry

    lax.fori_loop(0, _NSTEP, step, 0)
    plsc.subcore_barrier()
    pltpu.sync_copy(dacc.at[pl.ds(base, _RPT)],
                    degout.at[c, pl.ds(base, _RPT)])


_sc_deg = pl.kernel(
    _deg_body,
    out_type=jax.ShapeDtypeStruct((_NC, _NPAD, _DW), jnp.float32),
    mesh=_mesh,
    scratch_types=[
        pltpu.VMEM((_NSTEP, _B), jnp.int32),
        pltpu.VMEM((_B, _DW), jnp.float32),
        pltpu.VMEM_SHARED((_NPAD, _DW), jnp.float32),
    ],
)


def _row_spec():
    return pl.BlockSpec((_ROWBLK, _H), lambda i: (i, 0))


def _full_spec(shape):
    nd = len(shape)
    return pl.BlockSpec(shape, lambda i: (0,) * nd)


def _part_spec(core):
    return pl.BlockSpec((1, _ROWBLK, _H), lambda i, c=core: (c, i, 0))


def _deg_spec(core):
    return pl.BlockSpec((1, _ROWBLK, _DW), lambda i, c=core: (c, i, 0))


def _proj_body(x, wl, bl, wn, ws, b0, hn_o, hs_o):
    h = jnp.dot(x[...], wl[...], preferred_element_type=jnp.float32) + bl[...]
    hn_o[...] = jnp.dot(h, wn[...], preferred_element_type=jnp.float32)
    hs_o[...] = jnp.dot(h, ws[...], preferred_element_type=jnp.float32) + b0[...]


def _tc_proj(x, wl, bl, wn, ws, b0):
    return pl.pallas_call(
        _proj_body,
        grid=(_GRID,),
        in_specs=[_row_spec(), _full_spec((_H, _H)), _full_spec((1, _H)),
                  _full_spec((_H, _H)), _full_spec((_H, _H)),
                  _full_spec((1, _H))],
        out_specs=[_row_spec(), _row_spec()],
        out_shape=[jax.ShapeDtypeStruct((_N, _H), jnp.float32)] * 2,
    )(x, wl, bl, wn, ws, b0)


def _relu_mean(p0, p1, d0, d1, hs):
    deg = jnp.max(d0[0] + d1[0], axis=1, keepdims=True)
    deg = jnp.maximum(deg, 1.0)
    return jnp.maximum(hs[...] + (p0[0] + p1[0]) / deg, 0.0)


def _combine_body(p0, p1, d0, d1, hs, wn, ws, b, hn_o, hs_o):
    h = _relu_mean(p0, p1, d0, d1, hs)
    hn_o[...] = jnp.dot(h, wn[...], preferred_element_type=jnp.float32)
    hs_o[...] = jnp.dot(h, ws[...], preferred_element_type=jnp.float32) + b[...]


def _tc_combine(p, degp, hs, wn, ws, b):
    return pl.pallas_call(
        _combine_body,
        grid=(_GRID,),
        in_specs=[_part_spec(0), _part_spec(1), _deg_spec(0), _deg_spec(1),
                  _row_spec(), _full_spec((_H, _H)), _full_spec((_H, _H)),
                  _full_spec((1, _H))],
        out_specs=[_row_spec(), _row_spec()],
        out_shape=[jax.ShapeDtypeStruct((_N, _H), jnp.float32)] * 2,
    )(p, p, degp, degp, hs, wn, ws, b)


def _final_body(p0, p1, d0, d1, hs, wc, bc, out_o):
    h = _relu_mean(p0, p1, d0, d1, hs)
    out_o[...] = jnp.dot(h, wc[...], preferred_element_type=jnp.float32) + bc[...]


def _tc_final(p, degp, hs, wc, bc):
    return pl.pallas_call(
        _final_body,
        grid=(_GRID,),
        in_specs=[_part_spec(0), _part_spec(1), _deg_spec(0), _deg_spec(1),
                  _row_spec(), _full_spec((_H, _H)), _full_spec((1, _H))],
        out_specs=_row_spec(),
        out_shape=jax.ShapeDtypeStruct((_N, _H), jnp.float32),
    )(p, p, degp, degp, hs, wc, bc)


def kernel(x, edge_index, W_lin1, b_lin1, W_self, W_neigh, b_layers,
           W_cls, b_cls):
    src3 = edge_index[0].reshape(_NW, _NSTEP, _B)
    dst3 = edge_index[1].reshape(_NW, _NSTEP, _B)
    zrows = jnp.zeros((_RPT, _H), jnp.float32)
    dz = jnp.zeros((_RPT, _DW), jnp.float32)
    ones_in = jnp.ones((_B, _DW), jnp.float32)

    degp = _sc_deg(dst3, dz, ones_in)
    hn, hs = _tc_proj(x, W_lin1, b_lin1.reshape(1, _H),
                      W_neigh[0], W_self[0], b_layers[0].reshape(1, _H))
    p = _sc_agg(hn, src3, dst3, zrows)
    for l in (1, 2):
        hn, hs = _tc_combine(p, degp, hs, W_neigh[l], W_self[l],
                             b_layers[l].reshape(1, _H))
        p = _sc_agg(hn, src3, dst3, zrows)
    return _tc_final(p, degp, hs, W_cls, b_cls.reshape(1, _H))


# speedup vs baseline: 10.4862x; 1.6955x over previous
import jax
import jax.numpy as jnp
from jax import lax
from jax.experimental import pallas as pl
from jax.experimental.pallas import tpu as pltpu
from jax.experimental.pallas import tpu_sc as plsc

_N = 10000
_E = 320000
_H = 128
_NC = 2
_NS = 16
_NW = _NC * _NS
_EPW = _E // _NW
_B = 80
_NSTEP = _EPW // _B
_NB = 4
_ND = 8
_NPAD = 10240
_RPT = _NPAD // _NS
_DW = 128
_ROWBLK = 1000
_GRID = _N // _ROWBLK

_mesh = plsc.VectorSubcoreMesh(
    core_axis_name="c", subcore_axis_name="s",
    num_cores=_NC, num_subcores=_NS)


def _agg_body(hn, srcs, dsts, zrows, out, sidx, didx, rows,
              gsem, ssem, isem, dsem, acc):
    c = lax.axis_index("c")
    s = lax.axis_index("s")
    wid = c * _NS + s
    base = s * _RPT

    pltpu.sync_copy(zrows, acc.at[pl.ds(base, _RPT)])

    for k in range(3):
        pltpu.async_copy(srcs.at[wid, k], sidx.at[k], isem.at[k])
        pltpu.async_copy(dsts.at[wid, k], didx.at[k], dsem.at[k])
    for k in range(2):
        pltpu.make_async_copy(srcs.at[wid, 0], sidx.at[k], isem.at[k]).wait()
        pltpu.async_copy(hn.at[sidx.at[k]], rows.at[k], gsem.at[k])
    plsc.subcore_barrier()

    def step(i, carry):
        r = lax.rem(i, _NB)
        d = lax.rem(i, _ND)
        pltpu.make_async_copy(hn.at[sidx.at[0]], rows.at[r], gsem.at[r]).wait()
        pltpu.make_async_copy(dsts.at[wid, 0], didx.at[d], dsem.at[d]).wait()
        pltpu.async_copy(rows.at[r], acc.at[didx.at[d]], ssem.at[r], add=True)

        @pl.when(i >= 2)
        def _():
            r2 = lax.rem(i + 2, _NB)
            pltpu.make_async_copy(rows.at[r2], acc.at[didx.at[0]],
                                  ssem.at[r2]).wait()

        @pl.when(i < _NSTEP - 3)
        def _():
            k4 = lax.rem(i + 3, _NB)
            k8 = lax.rem(i + 3, _ND)
            pltpu.async_copy(srcs.at[wid, i + 3], sidx.at[k4], isem.at[k4])
            pltpu.async_copy(dsts.at[wid, i + 3], didx.at[k8], dsem.at[k8])

        @pl.when(i < _NSTEP - 2)
        def _():
            k4 = lax.rem(i + 2, _NB)
            pltpu.make_async_copy(srcs.at[wid, 0], sidx.at[k4],
                                  isem.at[k4]).wait()
            pltpu.async_copy(hn.at[sidx.at[k4]], rows.at[k4], gsem.at[k4])

        return carry

    lax.fori_loop(0, _NSTEP, step, 0)
    for i in (_NSTEP - 2, _NSTEP - 1):
        pltpu.make_async_copy(rows.at[i % _NB], acc.at[didx.at[0]],
                              ssem.at[i % _NB]).wait()
    plsc.subcore_barrier()

    pltpu.sync_copy(acc.at[pl.ds(base, _RPT)], out.at[c, pl.ds(base, _RPT)])


_sc_agg = pl.kernel(
    _agg_body,
    out_type=jax.ShapeDtypeStruct((_NC, _NPAD, _H), jnp.float32),
    mesh=_mesh,
    scratch_types=[
        pltpu.VMEM((_NB, _B), jnp.int32),
        pltpu.VMEM((_ND, _B), jnp.int32),
        pltpu.VMEM((_NB, _B, _H), jnp.float32),
        pltpu.SemaphoreType.DMA((_NB,)),
        pltpu.SemaphoreType.DMA((_NB,)),
        pltpu.SemaphoreType.DMA((_NB,)),
        pltpu.SemaphoreType.DMA((_ND,)),
        pltpu.VMEM_SHARED((_NPAD, _H), jnp.float32),
    ],
)


def _deg_body(dsts, dz, ones_in, degout, dst_v, ones_v, dacc, sem):
    c = lax.axis_index("c")
    s = lax.axis_index("s")
    wid = c * _NS + s
    base = s * _RPT

    pltpu.sync_copy(dz, dacc.at[pl.ds(base, _RPT)])
    pltpu.sync_copy(ones_in, ones_v)
    pltpu.sync_copy(dsts.at[wid], dst_v)
    plsc.subcore_barrier()

    def fire(i, carry):
        pltpu.async_copy(ones_v, dacc.at[dst_v.at[i]], sem, add=True)
        return carry

    lax.fori_loop(0, _NSTEP, fire, 0)

    def drain(i, carry):
        pltpu.make_async_copy(ones_v, dacc.at[dst_v.at[0]], sem).wait()
        return carry

    lax.fori_loop(0, _NSTEP, drain, 0)
    plsc.subcore_barrier()
    pltpu.sync_copy(dacc.at[pl.ds(base, _RPT)],
                    degout.at[c, pl.ds(base, _RPT)])


_sc_deg = pl.kernel(
    _deg_body,
    out_type=jax.ShapeDtypeStruct((_NC, _NPAD, _DW), jnp.float32),
    mesh=_mesh,
    scratch_types=[
        pltpu.VMEM((_NSTEP, _B), jnp.int32),
        pltpu.VMEM((_B, _DW), jnp.float32),
        pltpu.VMEM_SHARED((_NPAD, _DW), jnp.float32),
        pltpu.SemaphoreType.DMA,
    ],
)


def _row_spec():
    return pl.BlockSpec((_ROWBLK, _H), lambda i: (i, 0))


def _full_spec(shape):
    nd = len(shape)
    return pl.BlockSpec(shape, lambda i: (0,) * nd)


def _part_spec(core):
    return pl.BlockSpec((1, _ROWBLK, _H), lambda i, c=core: (c, i, 0))


def _deg_spec(core):
    return pl.BlockSpec((1, _ROWBLK, _DW), lambda i, c=core: (c, i, 0))


def _proj_body(x, wl, bl, wn, ws, b0, hn_o, hs_o):
    h = jnp.dot(x[...], wl[...], preferred_element_type=jnp.float32) + bl[...]
    hn_o[...] = jnp.dot(h, wn[...], preferred_element_type=jnp.float32)
    hs_o[...] = jnp.dot(h, ws[...], preferred_element_type=jnp.float32) + b0[...]


def _tc_proj(x, wl, bl, wn, ws, b0):
    return pl.pallas_call(
        _proj_body,
        grid=(_GRID,),
        in_specs=[_row_spec(), _full_spec((_H, _H)), _full_spec((1, _H)),
                  _full_spec((_H, _H)), _full_spec((_H, _H)),
                  _full_spec((1, _H))],
        out_specs=[_row_spec(), _row_spec()],
        out_shape=[jax.ShapeDtypeStruct((_N, _H), jnp.float32)] * 2,
    )(x, wl, bl, wn, ws, b0)


def _relu_mean(p0, p1, d0, d1, hs):
    deg = jnp.max(d0[0] + d1[0], axis=1, keepdims=True)
    deg = jnp.maximum(deg, 1.0)
    return jnp.maximum(hs[...] + (p0[0] + p1[0]) / deg, 0.0)


def _combine_body(p0, p1, d0, d1, hs, wn, ws, b, hn_o, hs_o):
    h = _relu_mean(p0, p1, d0, d1, hs)
    hn_o[...] = jnp.dot(h, wn[...], preferred_element_type=jnp.float32)
    hs_o[...] = jnp.dot(h, ws[...], preferred_element_type=jnp.float32) + b[...]


def _tc_combine(p, degp, hs, wn, ws, b):
    return pl.pallas_call(
        _combine_body,
        grid=(_GRID,),
        in_specs=[_part_spec(0), _part_spec(1), _deg_spec(0), _deg_spec(1),
                  _row_spec(), _full_spec((_H, _H)), _full_spec((_H, _H)),
                  _full_spec((1, _H))],
        out_specs=[_row_spec(), _row_spec()],
        out_shape=[jax.ShapeDtypeStruct((_N, _H), jnp.float32)] * 2,
    )(p, p, degp, degp, hs, wn, ws, b)


def _final_body(p0, p1, d0, d1, hs, wc, bc, out_o):
    h = _relu_mean(p0, p1, d0, d1, hs)
    out_o[...] = jnp.dot(h, wc[...], preferred_element_type=jnp.float32) + bc[...]


def _tc_final(p, degp, hs, wc, bc):
    return pl.pallas_call(
        _final_body,
        grid=(_GRID,),
        in_specs=[_part_spec(0), _part_spec(1), _deg_spec(0), _deg_spec(1),
                  _row_spec(), _full_spec((_H, _H)), _full_spec((1, _H))],
        out_specs=_row_spec(),
        out_shape=jax.ShapeDtypeStruct((_N, _H), jnp.float32),
    )(p, p, degp, degp, hs, wc, bc)


def kernel(x, edge_index, W_lin1, b_lin1, W_self, W_neigh, b_layers,
           W_cls, b_cls):
    src3 = edge_index[0].reshape(_NW, _NSTEP, _B)
    dst3 = edge_index[1].reshape(_NW, _NSTEP, _B)
    zrows = jnp.zeros((_RPT, _H), jnp.float32)
    dz = jnp.zeros((_RPT, _DW), jnp.float32)
    ones_in = jnp.ones((_B, _DW), jnp.float32)

    degp = _sc_deg(dst3, dz, ones_in)
    hn, hs = _tc_proj(x, W_lin1, b_lin1.reshape(1, _H),
                      W_neigh[0], W_self[0], b_layers[0].reshape(1, _H))
    p = _sc_agg(hn, src3, dst3, zrows)
    for l in (1, 2):
        hn, hs = _tc_combine(p, degp, hs, W_neigh[l], W_self[l],
                             b_layers[l].reshape(1, _H))
        p = _sc_agg(hn, src3, dst3, zrows)
    return _tc_final(p, degp, hs, W_cls, b_cls.reshape(1, _H))


# speedup vs baseline: 10.7543x; 1.0256x over previous
import jax
import jax.numpy as jnp
from jax import lax
from jax.experimental import pallas as pl
from jax.experimental.pallas import tpu as pltpu
from jax.experimental.pallas import tpu_sc as plsc

_N = 10000
_E = 320000
_H = 128
_NC = 2
_NS = 16
_NW = _NC * _NS
_EPW = _E // _NW
_B = 80
_NSTEP = _EPW // _B
_NB = 3
_NPAD = 10240
_RPT = _NPAD // _NS
_DW = 128
_ROWBLK = 1000
_GRID = _N // _ROWBLK

_mesh = plsc.VectorSubcoreMesh(
    core_axis_name="c", subcore_axis_name="s",
    num_cores=_NC, num_subcores=_NS)


def _agg_body(hn, srcs, dsts, out, sidx, didx, rows,
              gsem, ssem, isem, acc):
    c = lax.axis_index("c")
    s = lax.axis_index("s")
    wid = c * _NS + s
    base = s * _RPT

    pltpu.sync_copy(dsts.at[wid], didx)
    for k in range(3):
        pltpu.async_copy(srcs.at[wid, k], sidx.at[k], isem.at[k])
    for k in range(2):
        pltpu.make_async_copy(srcs.at[wid, 0], sidx.at[k], isem.at[k]).wait()
        pltpu.async_copy(hn.at[sidx.at[k]], rows.at[k], gsem.at[k])

    z16 = jnp.zeros((16,), jnp.float32)

    def zrow(r, carry):
        for j in range(_H // 16):
            rows[2, r, pl.ds(j * 16, 16)] = z16
        return carry

    lax.fori_loop(0, _B, zrow, 0)
    for k in range(_RPT // _B):
        pltpu.sync_copy(rows.at[2], acc.at[pl.ds(base + k * _B, _B)])
    plsc.subcore_barrier()

    def step(i, carry):
        r = lax.rem(i, _NB)
        pltpu.make_async_copy(hn.at[sidx.at[0]], rows.at[r], gsem.at[r]).wait()
        pltpu.async_copy(rows.at[r], acc.at[didx.at[i]], ssem.at[r], add=True)

        @pl.when(i >= 1)
        def _():
            r2 = lax.rem(i + 2, _NB)
            pltpu.make_async_copy(rows.at[r2], acc.at[didx.at[0]],
                                  ssem.at[r2]).wait()

        @pl.when(i < _NSTEP - 3)
        def _():
            kb = lax.rem(i + 3, _NB)
            pltpu.async_copy(srcs.at[wid, i + 3], sidx.at[kb], isem.at[kb])

        @pl.when(i < _NSTEP - 2)
        def _():
            kb = lax.rem(i + 2, _NB)
            pltpu.make_async_copy(srcs.at[wid, 0], sidx.at[kb],
                                  isem.at[kb]).wait()
            pltpu.async_copy(hn.at[sidx.at[kb]], rows.at[kb], gsem.at[kb])

        return carry

    lax.fori_loop(0, _NSTEP, step, 0)
    pltpu.make_async_copy(rows.at[(_NSTEP - 1) % _NB], acc.at[didx.at[0]],
                          ssem.at[(_NSTEP - 1) % _NB]).wait()
    plsc.subcore_barrier()

    pltpu.sync_copy(acc.at[pl.ds(base, _RPT)], out.at[c, pl.ds(base, _RPT)])


_sc_agg = pl.kernel(
    _agg_body,
    out_type=jax.ShapeDtypeStruct((_NC, _NPAD, _H), jnp.float32),
    mesh=_mesh,
    scratch_types=[
        pltpu.VMEM((_NB, _B), jnp.int32),
        pltpu.VMEM((_NSTEP, _B), jnp.int32),
        pltpu.VMEM((_NB, _B, _H), jnp.float32),
        pltpu.SemaphoreType.DMA((_NB,)),
        pltpu.SemaphoreType.DMA((_NB,)),
        pltpu.SemaphoreType.DMA((_NB,)),
        pltpu.VMEM_SHARED((_NPAD, _H), jnp.float32),
    ],
)


def _deg_body(dsts, dz, ones_in, degout, dst_v, ones_v, dacc, sem):
    c = lax.axis_index("c")
    s = lax.axis_index("s")
    wid = c * _NS + s
    base = s * _RPT

    pltpu.sync_copy(dz, dacc.at[pl.ds(base, _RPT)])
    pltpu.sync_copy(ones_in, ones_v)
    pltpu.sync_copy(dsts.at[wid], dst_v)
    plsc.subcore_barrier()

    def fire(i, carry):
        pltpu.async_copy(ones_v, dacc.at[dst_v.at[i]], sem, add=True)
        return carry

    lax.fori_loop(0, _NSTEP, fire, 0)

    def drain(i, carry):
        pltpu.make_async_copy(ones_v, dacc.at[dst_v.at[0]], sem).wait()
        return carry

    lax.fori_loop(0, _NSTEP, drain, 0)
    plsc.subcore_barrier()
    pltpu.sync_copy(dacc.at[pl.ds(base, _RPT)],
                    degout.at[c, pl.ds(base, _RPT)])


_sc_deg = pl.kernel(
    _deg_body,
    out_type=jax.ShapeDtypeStruct((_NC, _NPAD, _DW), jnp.float32),
    mesh=_mesh,
    scratch_types=[
        pltpu.VMEM((_NSTEP, _B), jnp.int32),
        pltpu.VMEM((_B, _DW), jnp.float32),
        pltpu.VMEM_SHARED((_NPAD, _DW), jnp.float32),
        pltpu.SemaphoreType.DMA,
    ],
)


def _row_spec():
    return pl.BlockSpec((_ROWBLK, _H), lambda i: (i, 0))


def _full_spec(shape):
    nd = len(shape)
    return pl.BlockSpec(shape, lambda i: (0,) * nd)


def _part_spec(core):
    return pl.BlockSpec((1, _ROWBLK, _H), lambda i, c=core: (c, i, 0))


def _deg_spec(core):
    return pl.BlockSpec((1, _ROWBLK, _DW), lambda i, c=core: (c, i, 0))


def _proj_body(x, wl, bl, wn, ws, b0, hn_o, hs_o):
    h = jnp.dot(x[...], wl[...], preferred_element_type=jnp.float32) + bl[...]
    hn_o[...] = jnp.dot(h, wn[...], preferred_element_type=jnp.float32)
    hs_o[...] = jnp.dot(h, ws[...], preferred_element_type=jnp.float32) + b0[...]


def _tc_proj(x, wl, bl, wn, ws, b0):
    return pl.pallas_call(
        _proj_body,
        grid=(_GRID,),
        in_specs=[_row_spec(), _full_spec((_H, _H)), _full_spec((1, _H)),
                  _full_spec((_H, _H)), _full_spec((_H, _H)),
                  _full_spec((1, _H))],
        out_specs=[_row_spec(), _row_spec()],
        out_shape=[jax.ShapeDtypeStruct((_N, _H), jnp.float32)] * 2,
    )(x, wl, bl, wn, ws, b0)


def _relu_mean(p0, p1, d0, d1, hs):
    deg = jnp.max(d0[0] + d1[0], axis=1, keepdims=True)
    deg = jnp.maximum(deg, 1.0)
    return jnp.maximum(hs[...] + (p0[0] + p1[0]) / deg, 0.0)


def _combine_body(p0, p1, d0, d1, hs, wn, ws, b, hn_o, hs_o):
    h = _relu_mean(p0, p1, d0, d1, hs)
    hn_o[...] = jnp.dot(h, wn[...], preferred_element_type=jnp.float32)
    hs_o[...] = jnp.dot(h, ws[...], preferred_element_type=jnp.float32) + b[...]


def _tc_combine(p, degp, hs, wn, ws, b):
    return pl.pallas_call(
        _combine_body,
        grid=(_GRID,),
        in_specs=[_part_spec(0), _part_spec(1), _deg_spec(0), _deg_spec(1),
                  _row_spec(), _full_spec((_H, _H)), _full_spec((_H, _H)),
                  _full_spec((1, _H))],
        out_specs=[_row_spec(), _row_spec()],
        out_shape=[jax.ShapeDtypeStruct((_N, _H), jnp.float32)] * 2,
    )(p, p, degp, degp, hs, wn, ws, b)


def _final_body(p0, p1, d0, d1, hs, wc, bc, out_o):
    h = _relu_mean(p0, p1, d0, d1, hs)
    out_o[...] = jnp.dot(h, wc[...], preferred_element_type=jnp.float32) + bc[...]


def _tc_final(p, degp, hs, wc, bc):
    return pl.pallas_call(
        _final_body,
        grid=(_GRID,),
        in_specs=[_part_spec(0), _part_spec(1), _deg_spec(0), _deg_spec(1),
                  _row_spec(), _full_spec((_H, _H)), _full_spec((1, _H))],
        out_specs=_row_spec(),
        out_shape=jax.ShapeDtypeStruct((_N, _H), jnp.float32),
    )(p, p, degp, degp, hs, wc, bc)


def kernel(x, edge_index, W_lin1, b_lin1, W_self, W_neigh, b_layers,
           W_cls, b_cls):
    src3 = edge_index[0].reshape(_NW, _NSTEP, _B)
    dst3 = edge_index[1].reshape(_NW, _NSTEP, _B)
    dz = jnp.zeros((_RPT, _DW), jnp.float32)
    ones_in = jnp.ones((_B, _DW), jnp.float32)

    degp = _sc_deg(dst3, dz, ones_in)
    hn, hs = _tc_proj(x, W_lin1, b_lin1.reshape(1, _H),
                      W_neigh[0], W_self[0], b_layers[0].reshape(1, _H))
    p = _sc_agg(hn, src3, dst3)
    for l in (1, 2):
        hn, hs = _tc_combine(p, degp, hs, W_neigh[l], W_self[l],
                             b_layers[l].reshape(1, _H))
        p = _sc_agg(hn, src3, dst3)
    return _tc_final(p, degp, hs, W_cls, b_cls.reshape(1, _H))


# speedup vs baseline: 10.8497x; 1.0089x over previous
import jax
import jax.numpy as jnp
from jax import lax
from jax.experimental import pallas as pl
from jax.experimental.pallas import tpu as pltpu
from jax.experimental.pallas import tpu_sc as plsc

_N = 10000
_E = 320000
_H = 128
_NC = 2
_NS = 16
_NW = _NC * _NS
_EPW = _E // _NW
_B = 80
_NSTEP = _EPW // _B
_NB = 3
_NPAD = 10240
_RPT = _NPAD // _NS
_DW = 128
_ROWBLK = 1000
_GRID = _N // _ROWBLK

_mesh = plsc.VectorSubcoreMesh(
    core_axis_name="c", subcore_axis_name="s",
    num_cores=_NC, num_subcores=_NS)


def _agg_body(hn, srcs, dsts, out, sidx, didx, rows,
              gsem, ssem, isem, acc):
    c = lax.axis_index("c")
    s = lax.axis_index("s")
    wid = c * _NS + s
    base = s * _RPT

    pltpu.sync_copy(dsts.at[wid], didx)
    for k in range(3):
        pltpu.async_copy(srcs.at[wid, k], sidx.at[k], isem.at[k])
    for k in range(2):
        pltpu.make_async_copy(srcs.at[wid, 0], sidx.at[k], isem.at[k]).wait()
        pltpu.async_copy(hn.at[sidx.at[k]], rows.at[k], gsem.at[k])

    z16 = jnp.zeros((16,), jnp.float32)

    def zrow(r, carry):
        for j in range(_H // 16):
            rows[2, r, pl.ds(j * 16, 16)] = z16
        return carry

    lax.fori_loop(0, _B, zrow, 0)
    for k in range(_RPT // _B):
        pltpu.sync_copy(rows.at[2], acc.at[pl.ds(base + k * _B, _B)])
    plsc.subcore_barrier()

    def step(i, carry):
        r = lax.rem(i, _NB)
        pltpu.make_async_copy(hn.at[sidx.at[0]], rows.at[r], gsem.at[r]).wait()
        pltpu.async_copy(rows.at[r], acc.at[didx.at[i]], ssem.at[r], add=True)

        @pl.when(i >= 1)
        def _():
            r2 = lax.rem(i + 2, _NB)
            pltpu.make_async_copy(rows.at[r2], acc.at[didx.at[0]],
                                  ssem.at[r2]).wait()

        @pl.when(i < _NSTEP - 3)
        def _():
            kb = lax.rem(i + 3, _NB)
            pltpu.async_copy(srcs.at[wid, i + 3], sidx.at[kb], isem.at[kb])

        @pl.when(i < _NSTEP - 2)
        def _():
            kb = lax.rem(i + 2, _NB)
            pltpu.make_async_copy(srcs.at[wid, 0], sidx.at[kb],
                                  isem.at[kb]).wait()
            pltpu.async_copy(hn.at[sidx.at[kb]], rows.at[kb], gsem.at[kb])

        return carry

    lax.fori_loop(0, _NSTEP, step, 0)
    pltpu.make_async_copy(rows.at[(_NSTEP - 1) % _NB], acc.at[didx.at[0]],
                          ssem.at[(_NSTEP - 1) % _NB]).wait()
    plsc.subcore_barrier()

    pltpu.sync_copy(acc.at[pl.ds(base, _RPT)], out.at[c, pl.ds(base, _RPT)])


_sc_agg = pl.kernel(
    _agg_body,
    out_type=jax.ShapeDtypeStruct((_NC, _NPAD, _H), jnp.float32),
    mesh=_mesh,
    scratch_types=[
        pltpu.VMEM((_NB, _B), jnp.int32),
        pltpu.VMEM((_NSTEP, _B), jnp.int32),
        pltpu.VMEM((_NB, _B, _H), jnp.float32),
        pltpu.SemaphoreType.DMA((_NB,)),
        pltpu.SemaphoreType.DMA((_NB,)),
        pltpu.SemaphoreType.DMA((_NB,)),
        pltpu.VMEM_SHARED((_NPAD, _H), jnp.float32),
    ],
)


def _deg_body(dsts, dz, ones_in, degout, dst_v, ones_v, dacc, sem):
    c = lax.axis_index("c")
    s = lax.axis_index("s")
    wid = c * _NS + s
    base = s * _RPT

    pltpu.sync_copy(dz, dacc.at[pl.ds(base, _RPT)])
    pltpu.sync_copy(ones_in, ones_v)
    pltpu.sync_copy(dsts.at[wid], dst_v)
    plsc.subcore_barrier()

    def fire(i, carry):
        pltpu.async_copy(ones_v, dacc.at[dst_v.at[i]], sem, add=True)
        return carry

    lax.fori_loop(0, _NSTEP, fire, 0)

    def drain(i, carry):
        pltpu.make_async_copy(ones_v, dacc.at[dst_v.at[0]], sem).wait()
        return carry

    lax.fori_loop(0, _NSTEP, drain, 0)
    plsc.subcore_barrier()
    pltpu.sync_copy(dacc.at[pl.ds(base, _RPT)],
                    degout.at[c, pl.ds(base, _RPT)])


_sc_deg = pl.kernel(
    _deg_body,
    out_type=jax.ShapeDtypeStruct((_NC, _NPAD, _DW), jnp.float32),
    mesh=_mesh,
    scratch_types=[
        pltpu.VMEM((_NSTEP, _B), jnp.int32),
        pltpu.VMEM((_B, _DW), jnp.float32),
        pltpu.VMEM_SHARED((_NPAD, _DW), jnp.float32),
        pltpu.SemaphoreType.DMA,
    ],
)


def _row_spec():
    return pl.BlockSpec((_ROWBLK, _H), lambda i: (i, 0))


def _full_spec(shape):
    nd = len(shape)
    return pl.BlockSpec(shape, lambda i: (0,) * nd)


def _part_spec(core):
    return pl.BlockSpec((1, _ROWBLK, _H), lambda i, c=core: (c, i, 0))


def _deg_spec(core):
    return pl.BlockSpec((1, _ROWBLK, _DW), lambda i, c=core: (c, i, 0))


def _rdeg_spec():
    return pl.BlockSpec((_ROWBLK, 1), lambda i: (i, 0))


def _proj_body(x, wl, bl, wn, ws, b0, d0, d1, hn_o, hs_o, rdeg_o):
    h = jnp.dot(x[...], wl[...], preferred_element_type=jnp.float32) + bl[...]
    hn_o[...] = jnp.dot(h, wn[...], preferred_element_type=jnp.float32)
    hs_o[...] = jnp.dot(h, ws[...], preferred_element_type=jnp.float32) + b0[...]
    deg = jnp.max(d0[0] + d1[0], axis=1, keepdims=True)
    rdeg_o[...] = 1.0 / jnp.maximum(deg, 1.0)


def _tc_proj(x, wl, bl, wn, ws, b0, degp):
    return pl.pallas_call(
        _proj_body,
        grid=(_GRID,),
        in_specs=[_row_spec(), _full_spec((_H, _H)), _full_spec((1, _H)),
                  _full_spec((_H, _H)), _full_spec((_H, _H)),
                  _full_spec((1, _H)), _deg_spec(0), _deg_spec(1)],
        out_specs=[_row_spec(), _row_spec(), _rdeg_spec()],
        out_shape=[jax.ShapeDtypeStruct((_N, _H), jnp.float32),
                   jax.ShapeDtypeStruct((_N, _H), jnp.float32),
                   jax.ShapeDtypeStruct((_N, 1), jnp.float32)],
    )(x, wl, bl, wn, ws, b0, degp, degp)


def _relu_mean(p0, p1, rdeg, hs):
    return jnp.maximum(hs[...] + (p0[0] + p1[0]) * rdeg[...], 0.0)


def _combine_body(p0, p1, rdeg, hs, wn, ws, b, hn_o, hs_o):
    h = _relu_mean(p0, p1, rdeg, hs)
    hn_o[...] = jnp.dot(h, wn[...], preferred_element_type=jnp.float32)
    hs_o[...] = jnp.dot(h, ws[...], preferred_element_type=jnp.float32) + b[...]


def _tc_combine(p, rdeg, hs, wn, ws, b):
    return pl.pallas_call(
        _combine_body,
        grid=(_GRID,),
        in_specs=[_part_spec(0), _part_spec(1), _rdeg_spec(),
                  _row_spec(), _full_spec((_H, _H)), _full_spec((_H, _H)),
                  _full_spec((1, _H))],
        out_specs=[_row_spec(), _row_spec()],
        out_shape=[jax.ShapeDtypeStruct((_N, _H), jnp.float32)] * 2,
    )(p, p, rdeg, hs, wn, ws, b)


def _final_body(p0, p1, rdeg, hs, wc, bc, out_o):
    h = _relu_mean(p0, p1, rdeg, hs)
    out_o[...] = jnp.dot(h, wc[...], preferred_element_type=jnp.float32) + bc[...]


def _tc_final(p, rdeg, hs, wc, bc):
    return pl.pallas_call(
        _final_body,
        grid=(_GRID,),
        in_specs=[_part_spec(0), _part_spec(1), _rdeg_spec(),
                  _row_spec(), _full_spec((_H, _H)), _full_spec((1, _H))],
        out_specs=_row_spec(),
        out_shape=jax.ShapeDtypeStruct((_N, _H), jnp.float32),
    )(p, p, rdeg, hs, wc, bc)


def kernel(x, edge_index, W_lin1, b_lin1, W_self, W_neigh, b_layers,
           W_cls, b_cls):
    src3 = edge_index[0].reshape(_NW, _NSTEP, _B)
    dst3 = edge_index[1].reshape(_NW, _NSTEP, _B)
    dz = jnp.zeros((_RPT, _DW), jnp.float32)
    ones_in = jnp.ones((_B, _DW), jnp.float32)

    degp = _sc_deg(dst3, dz, ones_in)
    hn, hs, rdeg = _tc_proj(x, W_lin1, b_lin1.reshape(1, _H),
                            W_neigh[0], W_self[0],
                            b_layers[0].reshape(1, _H), degp)
    p = _sc_agg(hn, src3, dst3)
    for l in (1, 2):
        hn, hs = _tc_combine(p, rdeg, hs, W_neigh[l], W_self[l],
                             b_layers[l].reshape(1, _H))
        p = _sc_agg(hn, src3, dst3)
    return _tc_final(p, rdeg, hs, W_cls, b_cls.reshape(1, _H))


# speedup vs baseline: 11.0791x; 1.0211x over previous
import jax
import jax.numpy as jnp
from jax import lax
from jax.experimental import pallas as pl
from jax.experimental.pallas import tpu as pltpu
from jax.experimental.pallas import tpu_sc as plsc

_N = 10000
_E = 320000
_H = 128
_NC = 2
_NS = 16
_NW = _NC * _NS
_EPW = _E // _NW
_B = 80
_NSTEP = _EPW // _B
_NB = 3
_NPAD = 10240
_RPT = _NPAD // _NS
_DW = 128
_ROWBLK = 2000
_GRID = _N // _ROWBLK

_mesh = plsc.VectorSubcoreMesh(
    core_axis_name="c", subcore_axis_name="s",
    num_cores=_NC, num_subcores=_NS)


def _agg_body(hn, srcs, dsts, out, sidx, didx, rows,
              gsem, ssem, isem, acc):
    c = lax.axis_index("c")
    s = lax.axis_index("s")
    wid = c * _NS + s
    base = s * _RPT

    pltpu.sync_copy(dsts.at[wid], didx)
    for k in range(3):
        pltpu.async_copy(srcs.at[wid, k], sidx.at[k], isem.at[k])
    for k in range(2):
        pltpu.make_async_copy(srcs.at[wid, 0], sidx.at[k], isem.at[k]).wait()
        pltpu.async_copy(hn.at[sidx.at[k]], rows.at[k], gsem.at[k])

    z16 = jnp.zeros((16,), jnp.float32)

    def zrow(r, carry):
        for j in range(_H // 16):
            rows[2, r, pl.ds(j * 16, 16)] = z16
        return carry

    lax.fori_loop(0, _B, zrow, 0)
    for k in range(_RPT // _B):
        pltpu.sync_copy(rows.at[2], acc.at[pl.ds(base + k * _B, _B)])
    plsc.subcore_barrier()

    def step(i, carry):
        r = lax.rem(i, _NB)
        pltpu.make_async_copy(hn.at[sidx.at[0]], rows.at[r], gsem.at[r]).wait()
        pltpu.async_copy(rows.at[r], acc.at[didx.at[i]], ssem.at[r], add=True)

        @pl.when(i >= 1)
        def _():
            r2 = lax.rem(i + 2, _NB)
            pltpu.make_async_copy(rows.at[r2], acc.at[didx.at[0]],
                                  ssem.at[r2]).wait()

        @pl.when(i < _NSTEP - 3)
        def _():
            kb = lax.rem(i + 3, _NB)
            pltpu.async_copy(srcs.at[wid, i + 3], sidx.at[kb], isem.at[kb])

        @pl.when(i < _NSTEP - 2)
        def _():
            kb = lax.rem(i + 2, _NB)
            pltpu.make_async_copy(srcs.at[wid, 0], sidx.at[kb],
                                  isem.at[kb]).wait()
            pltpu.async_copy(hn.at[sidx.at[kb]], rows.at[kb], gsem.at[kb])

        return carry

    lax.fori_loop(0, _NSTEP, step, 0)
    pltpu.make_async_copy(rows.at[(_NSTEP - 1) % _NB], acc.at[didx.at[0]],
                          ssem.at[(_NSTEP - 1) % _NB]).wait()
    plsc.subcore_barrier()

    pltpu.sync_copy(acc.at[pl.ds(base, _RPT)], out.at[c, pl.ds(base, _RPT)])


_sc_agg = pl.kernel(
    _agg_body,
    out_type=jax.ShapeDtypeStruct((_NC, _NPAD, _H), jnp.float32),
    mesh=_mesh,
    scratch_types=[
        pltpu.VMEM((_NB, _B), jnp.int32),
        pltpu.VMEM((_NSTEP, _B), jnp.int32),
        pltpu.VMEM((_NB, _B, _H), jnp.float32),
        pltpu.SemaphoreType.DMA((_NB,)),
        pltpu.SemaphoreType.DMA((_NB,)),
        pltpu.SemaphoreType.DMA((_NB,)),
        pltpu.VMEM_SHARED((_NPAD, _H), jnp.float32),
    ],
)


def _deg_body(dsts, dz, ones_in, degout, dst_v, ones_v, dacc, sem):
    c = lax.axis_index("c")
    s = lax.axis_index("s")
    wid = c * _NS + s
    base = s * _RPT

    pltpu.sync_copy(dz, dacc.at[pl.ds(base, _RPT)])
    pltpu.sync_copy(ones_in, ones_v)
    pltpu.sync_copy(dsts.at[wid], dst_v)
    plsc.subcore_barrier()

    def fire(i, carry):
        pltpu.async_copy(ones_v, dacc.at[dst_v.at[i]], sem, add=True)
        return carry

    lax.fori_loop(0, _NSTEP, fire, 0)

    def drain(i, carry):
        pltpu.make_async_copy(ones_v, dacc.at[dst_v.at[0]], sem).wait()
        return carry

    lax.fori_loop(0, _NSTEP, drain, 0)
    plsc.subcore_barrier()
    pltpu.sync_copy(dacc.at[pl.ds(base, _RPT)],
                    degout.at[c, pl.ds(base, _RPT)])


_sc_deg = pl.kernel(
    _deg_body,
    out_type=jax.ShapeDtypeStruct((_NC, _NPAD, _DW), jnp.float32),
    mesh=_mesh,
    scratch_types=[
        pltpu.VMEM((_NSTEP, _B), jnp.int32),
        pltpu.VMEM((_B, _DW), jnp.float32),
        pltpu.VMEM_SHARED((_NPAD, _DW), jnp.float32),
        pltpu.SemaphoreType.DMA,
    ],
)


def _row_spec():
    return pl.BlockSpec((_ROWBLK, _H), lambda i: (i, 0))


def _full_spec(shape):
    nd = len(shape)
    return pl.BlockSpec(shape, lambda i: (0,) * nd)


def _part_spec(core):
    return pl.BlockSpec((1, _ROWBLK, _H), lambda i, c=core: (c, i, 0))


def _deg_spec(core):
    return pl.BlockSpec((1, _ROWBLK, _DW), lambda i, c=core: (c, i, 0))


def _rdeg_spec():
    return pl.BlockSpec((_ROWBLK, 1), lambda i: (i, 0))


def _proj_body(x, wl, bl, wn, ws, b0, d0, d1, hn_o, hs_o, rdeg_o):
    h = jnp.dot(x[...], wl[...], preferred_element_type=jnp.float32) + bl[...]
    hn_o[...] = jnp.dot(h, wn[...], preferred_element_type=jnp.float32)
    hs_o[...] = jnp.dot(h, ws[...], preferred_element_type=jnp.float32) + b0[...]
    deg = jnp.max(d0[0] + d1[0], axis=1, keepdims=True)
    rdeg_o[...] = 1.0 / jnp.maximum(deg, 1.0)


def _tc_proj(x, wl, bl, wn, ws, b0, degp):
    return pl.pallas_call(
        _proj_body,
        grid=(_GRID,),
        in_specs=[_row_spec(), _full_spec((_H, _H)), _full_spec((1, _H)),
                  _full_spec((_H, _H)), _full_spec((_H, _H)),
                  _full_spec((1, _H)), _deg_spec(0), _deg_spec(1)],
        out_specs=[_row_spec(), _row_spec(), _rdeg_spec()],
        out_shape=[jax.ShapeDtypeStruct((_N, _H), jnp.float32),
                   jax.ShapeDtypeStruct((_N, _H), jnp.float32),
                   jax.ShapeDtypeStruct((_N, 1), jnp.float32)],
    )(x, wl, bl, wn, ws, b0, degp, degp)


def _relu_mean(p0, p1, rdeg, hs):
    return jnp.maximum(hs[...] + (p0[0] + p1[0]) * rdeg[...], 0.0)


def _combine_body(p0, p1, rdeg, hs, wn, ws, b, hn_o, hs_o):
    h = _relu_mean(p0, p1, rdeg, hs)
    hn_o[...] = jnp.dot(h, wn[...], preferred_element_type=jnp.float32)
    hs_o[...] = jnp.dot(h, ws[...], preferred_element_type=jnp.float32) + b[...]


def _tc_combine(p, rdeg, hs, wn, ws, b):
    return pl.pallas_call(
        _combine_body,
        grid=(_GRID,),
        in_specs=[_part_spec(0), _part_spec(1), _rdeg_spec(),
                  _row_spec(), _full_spec((_H, _H)), _full_spec((_H, _H)),
                  _full_spec((1, _H))],
        out_specs=[_row_spec(), _row_spec()],
        out_shape=[jax.ShapeDtypeStruct((_N, _H), jnp.float32)] * 2,
    )(p, p, rdeg, hs, wn, ws, b)


def _final_body(p0, p1, rdeg, hs, wc, bc, out_o):
    h = _relu_mean(p0, p1, rdeg, hs)
    out_o[...] = jnp.dot(h, wc[...], preferred_element_type=jnp.float32) + bc[...]


def _tc_final(p, rdeg, hs, wc, bc):
    return pl.pallas_call(
        _final_body,
        grid=(_GRID,),
        in_specs=[_part_spec(0), _part_spec(1), _rdeg_spec(),
                  _row_spec(), _full_spec((_H, _H)), _full_spec((1, _H))],
        out_specs=_row_spec(),
        out_shape=jax.ShapeDtypeStruct((_N, _H), jnp.float32),
    )(p, p, rdeg, hs, wc, bc)


def kernel(x, edge_index, W_lin1, b_lin1, W_self, W_neigh, b_layers,
           W_cls, b_cls):
    src3 = edge_index[0].reshape(_NW, _NSTEP, _B)
    dst3 = edge_index[1].reshape(_NW, _NSTEP, _B)
    dz = jnp.zeros((_RPT, _DW), jnp.float32)
    ones_in = jnp.ones((_B, _DW), jnp.float32)

    degp = _sc_deg(dst3, dz, ones_in)
    hn, hs, rdeg = _tc_proj(x, W_lin1, b_lin1.reshape(1, _H),
                            W_neigh[0], W_self[0],
                            b_layers[0].reshape(1, _H), degp)
    p = _sc_agg(hn, src3, dst3)
    for l in (1, 2):
        hn, hs = _tc_combine(p, rdeg, hs, W_neigh[l], W_self[l],
                             b_layers[l].reshape(1, _H))
        p = _sc_agg(hn, src3, dst3)
    return _tc_final(p, rdeg, hs, W_cls, b_cls.reshape(1, _H))


# speedup vs baseline: 11.1762x; 1.0088x over previous
import jax
import jax.numpy as jnp
from jax import lax
from jax.experimental import pallas as pl
from jax.experimental.pallas import tpu as pltpu
from jax.experimental.pallas import tpu_sc as plsc

_N = 10000
_E = 320000
_H = 128
_NC = 2
_NS = 16
_NW = _NC * _NS
_EPW = _E // _NW
_B = 80
_NSTEP = _EPW // _B
_NB = 3
_NPAD = 10240
_RPT = _NPAD // _NS
_DW = 128
_ROWBLK = 5000
_GRID = _N // _ROWBLK

_mesh = plsc.VectorSubcoreMesh(
    core_axis_name="c", subcore_axis_name="s",
    num_cores=_NC, num_subcores=_NS)


def _agg_body(hn, srcs, dsts, out, sidx, didx, rows,
              gsem, ssem, isem, acc):
    c = lax.axis_index("c")
    s = lax.axis_index("s")
    wid = c * _NS + s
    base = s * _RPT

    pltpu.sync_copy(dsts.at[wid], didx)
    for k in range(3):
        pltpu.async_copy(srcs.at[wid, k], sidx.at[k], isem.at[k])
    for k in range(2):
        pltpu.make_async_copy(srcs.at[wid, 0], sidx.at[k], isem.at[k]).wait()
        pltpu.async_copy(hn.at[sidx.at[k]], rows.at[k], gsem.at[k])

    z16 = jnp.zeros((16,), jnp.float32)

    def zrow(r, carry):
        for j in range(_H // 16):
            rows[2, r, pl.ds(j * 16, 16)] = z16
        return carry

    lax.fori_loop(0, _B, zrow, 0)
    for k in range(_RPT // _B):
        pltpu.sync_copy(rows.at[2], acc.at[pl.ds(base + k * _B, _B)])
    plsc.subcore_barrier()

    def step(i, carry):
        r = lax.rem(i, _NB)
        pltpu.make_async_copy(hn.at[sidx.at[0]], rows.at[r], gsem.at[r]).wait()
        pltpu.async_copy(rows.at[r], acc.at[didx.at[i]], ssem.at[r], add=True)

        @pl.when(i >= 1)
        def _():
            r2 = lax.rem(i + 2, _NB)
            pltpu.make_async_copy(rows.at[r2], acc.at[didx.at[0]],
                                  ssem.at[r2]).wait()

        @pl.when(i < _NSTEP - 3)
        def _():
            kb = lax.rem(i + 3, _NB)
            pltpu.async_copy(srcs.at[wid, i + 3], sidx.at[kb], isem.at[kb])

        @pl.when(i < _NSTEP - 2)
        def _():
            kb = lax.rem(i + 2, _NB)
            pltpu.make_async_copy(srcs.at[wid, 0], sidx.at[kb],
                                  isem.at[kb]).wait()
            pltpu.async_copy(hn.at[sidx.at[kb]], rows.at[kb], gsem.at[kb])

        return carry

    lax.fori_loop(0, _NSTEP, step, 0)
    pltpu.make_async_copy(rows.at[(_NSTEP - 1) % _NB], acc.at[didx.at[0]],
                          ssem.at[(_NSTEP - 1) % _NB]).wait()
    plsc.subcore_barrier()

    pltpu.sync_copy(acc.at[pl.ds(base, _RPT)], out.at[c, pl.ds(base, _RPT)])


_sc_agg = pl.kernel(
    _agg_body,
    out_type=jax.ShapeDtypeStruct((_NC, _NPAD, _H), jnp.float32),
    mesh=_mesh,
    scratch_types=[
        pltpu.VMEM((_NB, _B), jnp.int32),
        pltpu.VMEM((_NSTEP, _B), jnp.int32),
        pltpu.VMEM((_NB, _B, _H), jnp.float32),
        pltpu.SemaphoreType.DMA((_NB,)),
        pltpu.SemaphoreType.DMA((_NB,)),
        pltpu.SemaphoreType.DMA((_NB,)),
        pltpu.VMEM_SHARED((_NPAD, _H), jnp.float32),
    ],
)


def _deg_body(dsts, dz, ones_in, degout, dst_v, ones_v, dacc, sem):
    c = lax.axis_index("c")
    s = lax.axis_index("s")
    wid = c * _NS + s
    base = s * _RPT

    pltpu.sync_copy(dz, dacc.at[pl.ds(base, _RPT)])
    pltpu.sync_copy(ones_in, ones_v)
    pltpu.sync_copy(dsts.at[wid], dst_v)
    plsc.subcore_barrier()

    def fire(i, carry):
        pltpu.async_copy(ones_v, dacc.at[dst_v.at[i]], sem, add=True)
        return carry

    lax.fori_loop(0, _NSTEP, fire, 0)

    def drain(i, carry):
        pltpu.make_async_copy(ones_v, dacc.at[dst_v.at[0]], sem).wait()
        return carry

    lax.fori_loop(0, _NSTEP, drain, 0)
    plsc.subcore_barrier()
    pltpu.sync_copy(dacc.at[pl.ds(base, _RPT)],
                    degout.at[c, pl.ds(base, _RPT)])


_sc_deg = pl.kernel(
    _deg_body,
    out_type=jax.ShapeDtypeStruct((_NC, _NPAD, _DW), jnp.float32),
    mesh=_mesh,
    scratch_types=[
        pltpu.VMEM((_NSTEP, _B), jnp.int32),
        pltpu.VMEM((_B, _DW), jnp.float32),
        pltpu.VMEM_SHARED((_NPAD, _DW), jnp.float32),
        pltpu.SemaphoreType.DMA,
    ],
)


def _row_spec():
    return pl.BlockSpec((_ROWBLK, _H), lambda i: (i, 0))


def _full_spec(shape):
    nd = len(shape)
    return pl.BlockSpec(shape, lambda i: (0,) * nd)


def _part_spec(core):
    return pl.BlockSpec((1, _ROWBLK, _H), lambda i, c=core: (c, i, 0))


def _deg_spec(core):
    return pl.BlockSpec((1, _ROWBLK, _DW), lambda i, c=core: (c, i, 0))


def _rdeg_spec():
    return pl.BlockSpec((_ROWBLK, 1), lambda i: (i, 0))


def _proj_body(x, wl, bl, wn, ws, b0, d0, d1, hn_o, hs_o, rdeg_o):
    h = jnp.dot(x[...], wl[...], preferred_element_type=jnp.float32) + bl[...]
    hn_o[...] = jnp.dot(h, wn[...], preferred_element_type=jnp.float32)
    hs_o[...] = jnp.dot(h, ws[...], preferred_element_type=jnp.float32) + b0[...]
    deg = jnp.max(d0[0] + d1[0], axis=1, keepdims=True)
    rdeg_o[...] = 1.0 / jnp.maximum(deg, 1.0)


def _tc_proj(x, wl, bl, wn, ws, b0, degp):
    return pl.pallas_call(
        _proj_body,
        grid=(_GRID,),
        in_specs=[_row_spec(), _full_spec((_H, _H)), _full_spec((1, _H)),
                  _full_spec((_H, _H)), _full_spec((_H, _H)),
                  _full_spec((1, _H)), _deg_spec(0), _deg_spec(1)],
        out_specs=[_row_spec(), _row_spec(), _rdeg_spec()],
        out_shape=[jax.ShapeDtypeStruct((_N, _H), jnp.float32),
                   jax.ShapeDtypeStruct((_N, _H), jnp.float32),
                   jax.ShapeDtypeStruct((_N, 1), jnp.float32)],
    )(x, wl, bl, wn, ws, b0, degp, degp)


def _relu_mean(p0, p1, rdeg, hs):
    return jnp.maximum(hs[...] + (p0[0] + p1[0]) * rdeg[...], 0.0)


def _combine_body(p0, p1, rdeg, hs, wn, ws, b, hn_o, hs_o):
    h = _relu_mean(p0, p1, rdeg, hs)
    hn_o[...] = jnp.dot(h, wn[...], preferred_element_type=jnp.float32)
    hs_o[...] = jnp.dot(h, ws[...], preferred_element_type=jnp.float32) + b[...]


def _tc_combine(p, rdeg, hs, wn, ws, b):
    return pl.pallas_call(
        _combine_body,
        grid=(_GRID,),
        in_specs=[_part_spec(0), _part_spec(1), _rdeg_spec(),
                  _row_spec(), _full_spec((_H, _H)), _full_spec((_H, _H)),
                  _full_spec((1, _H))],
        out_specs=[_row_spec(), _row_spec()],
        out_shape=[jax.ShapeDtypeStruct((_N, _H), jnp.float32)] * 2,
    )(p, p, rdeg, hs, wn, ws, b)


def _final_body(p0, p1, rdeg, hs, wc, bc, out_o):
    h = _relu_mean(p0, p1, rdeg, hs)
    out_o[...] = jnp.dot(h, wc[...], preferred_element_type=jnp.float32) + bc[...]


def _tc_final(p, rdeg, hs, wc, bc):
    return pl.pallas_call(
        _final_body,
        grid=(_GRID,),
        in_specs=[_part_spec(0), _part_spec(1), _rdeg_spec(),
                  _row_spec(), _full_spec((_H, _H)), _full_spec((1, _H))],
        out_specs=_row_spec(),
        out_shape=jax.ShapeDtypeStruct((_N, _H), jnp.float32),
    )(p, p, rdeg, hs, wc, bc)


def kernel(x, edge_index, W_lin1, b_lin1, W_self, W_neigh, b_layers,
           W_cls, b_cls):
    src3 = edge_index[0].reshape(_NW, _NSTEP, _B)
    dst3 = edge_index[1].reshape(_NW, _NSTEP, _B)
    dz = jnp.zeros((_RPT, _DW), jnp.float32)
    ones_in = jnp.ones((_B, _DW), jnp.float32)

    degp = _sc_deg(dst3, dz, ones_in)
    hn, hs, rdeg = _tc_proj(x, W_lin1, b_lin1.reshape(1, _H),
                            W_neigh[0], W_self[0],
                            b_layers[0].reshape(1, _H), degp)
    p = _sc_agg(hn, src3, dst3)
    for l in (1, 2):
        hn, hs = _tc_combine(p, rdeg, hs, W_neigh[l], W_self[l],
                             b_layers[l].reshape(1, _H))
        p = _sc_agg(hn, src3, dst3)
    return _tc_final(p, rdeg, hs, W_cls, b_cls.reshape(1, _H))
